# baseline (speedup 1.0000x reference)
import functools
import jax
import jax.numpy as jnp
from jax import lax
from jax.experimental import pallas as pl
from jax.experimental.pallas import tpu as pltpu, tpu_sc as plsc

D_MODEL = 768
VECS = D_MODEL // 16
R = 64


def _make_sc(B, S, D):
    NW = 32
    s_per_w = S // NW
    n_chunks = s_per_w // R
    mesh = plsc.VectorSubcoreMesh(core_axis_name="c", subcore_axis_name="s")

    @functools.partial(
        pl.kernel,
        mesh=mesh,
        out_type=jax.ShapeDtypeStruct((B * S * D,), jnp.float32),
        scratch_types=[
            pltpu.VMEM((R * D,), jnp.float32),
            pltpu.VMEM((R * D,), jnp.float32),
        ],
    )
    def k(x_hbm, wpe_hbm, out_hbm, w_buf, x_buf):
        wid = lax.axis_index("s") * 2 + lax.axis_index("c")
        s0 = wid * s_per_w

        def chunk_body(ci, _):
            w_off = (s0 + ci * R) * D
            pltpu.sync_copy(wpe_hbm.at[pl.ds(w_off, R * D)], w_buf)

            def batch_body(b, _):
                x_off = (b * S + s0 + ci * R) * D
                pltpu.sync_copy(x_hbm.at[pl.ds(x_off, R * D)], x_buf)

                def add_body(i, _):
                    x_buf[pl.ds(i * 16, 16)] += w_buf[pl.ds(i * 16, 16)]
                    return 0

                lax.fori_loop(0, R * VECS, add_body, 0, unroll=8)
                pltpu.sync_copy(x_buf, out_hbm.at[pl.ds(x_off, R * D)])
                return 0

            lax.fori_loop(0, B, batch_body, 0)
            return 0

        lax.fori_loop(0, n_chunks, chunk_body, 0)

    return k


def kernel(x, wpe):
    B, S, D = x.shape
    out = _make_sc(B, S, D)(x.reshape(-1), wpe.reshape(-1))
    return out.reshape(B, S, D)


# speedup vs baseline: 1.5560x; 1.5560x over previous
import functools
import jax
import jax.numpy as jnp
from jax import lax
from jax.experimental import pallas as pl
from jax.experimental.pallas import tpu as pltpu, tpu_sc as plsc

NW = 32
RX = 16


def _make_sc(B, S, D):
    s_per_w = S // NW
    steps_per_b = s_per_w // RX
    n_steps = B * steps_per_b
    mesh = plsc.VectorSubcoreMesh(core_axis_name="c", subcore_axis_name="s")

    @functools.partial(
        pl.kernel,
        mesh=mesh,
        out_type=jax.ShapeDtypeStruct((B * S * D,), jnp.float32),
        scratch_types=[
            pltpu.VMEM((s_per_w * D,), jnp.float32),
            pltpu.VMEM((RX * D,), jnp.float32),
            pltpu.VMEM((RX * D,), jnp.float32),
            pltpu.SemaphoreType.DMA,
            pltpu.SemaphoreType.DMA,
            pltpu.SemaphoreType.DMA,
            pltpu.SemaphoreType.DMA,
        ],
    )
    def k(x_hbm, wpe_hbm, out_hbm, w_buf, xb0, xb1, si0, si1, so0, so1):
        wid = lax.axis_index("s") * 2 + lax.axis_index("c")
        s0 = wid * s_per_w
        xbufs = (xb0, xb1)
        in_sems = (si0, si1)
        out_sems = (so0, so1)

        pltpu.sync_copy(wpe_hbm.at[pl.ds(s0 * D, s_per_w * D)], w_buf)

        def x_slice(t):
            b, j = divmod(t, steps_per_b)
            return pl.ds((b * S + s0 + j * RX) * D, RX * D)

        in_descs = [None] * n_steps
        out_descs = [None] * n_steps

        in_descs[0] = pltpu.async_copy(x_hbm.at[x_slice(0)], xbufs[0],
                                       in_sems[0])
        for t in range(n_steps):
            p = t % 2
            if t >= 1:
                out_descs[t - 1].wait()
            if t + 1 < n_steps:
                in_descs[t + 1] = pltpu.async_copy(
                    x_hbm.at[x_slice(t + 1)], xbufs[(t + 1) % 2],
                    in_sems[(t + 1) % 2])
            in_descs[t].wait()

            j = t % steps_per_b
            xb = xbufs[p]

            @plsc.parallel_loop(0, RX * D // 16, 1, unroll=8)
            def _(i):
                xb[pl.ds(i * 16, 16)] += w_buf[pl.ds(j * RX * D + i * 16, 16)]

            out_descs[t] = pltpu.async_copy(xb, out_hbm.at[x_slice(t)],
                                            out_sems[p])
        out_descs[n_steps - 1].wait()

    return k


def kernel(x, wpe):
    B, S, D = x.shape
    out = _make_sc(B, S, D)(x.reshape(-1), wpe.reshape(-1))
    return out.reshape(B, S, D)


# speedup vs baseline: 3.6417x; 2.3405x over previous
import functools
import jax
import jax.numpy as jnp
from jax import lax
from jax.experimental import pallas as pl
from jax.experimental.pallas import tpu as pltpu, tpu_sc as plsc

NW = 32
RX = 16


def _make_sc(B, S, D):
    s_per_w = S // NW
    steps_per_b = s_per_w // RX
    n_steps = B * steps_per_b
    vecs = D // 16
    mesh = plsc.VectorSubcoreMesh(core_axis_name="c", subcore_axis_name="s")

    @functools.partial(
        pl.kernel,
        mesh=mesh,
        out_type=jax.ShapeDtypeStruct((B, S, D), jnp.float32),
        scratch_types=[
            pltpu.VMEM((s_per_w, D), jnp.float32),
            pltpu.VMEM((RX, D), jnp.float32),
            pltpu.VMEM((RX, D), jnp.float32),
            pltpu.SemaphoreType.DMA,
            pltpu.SemaphoreType.DMA,
            pltpu.SemaphoreType.DMA,
            pltpu.SemaphoreType.DMA,
        ],
    )
    def k(x_hbm, wpe_hbm, out_hbm, w_buf, xb0, xb1, si0, si1, so0, so1):
        wid = lax.axis_index("s") * 2 + lax.axis_index("c")
        s0 = wid * s_per_w
        xbufs = (xb0, xb1)
        in_sems = (si0, si1)
        out_sems = (so0, so1)

        pltpu.sync_copy(wpe_hbm.at[pl.ds(s0, s_per_w)], w_buf)

        def x_slice(t):
            b, j = divmod(t, steps_per_b)
            return (b, pl.ds(s0 + j * RX, RX))

        in_descs = [None] * n_steps
        out_descs = [None] * n_steps

        in_descs[0] = pltpu.async_copy(x_hbm.at[x_slice(0)], xbufs[0],
                                       in_sems[0])
        for t in range(n_steps):
            p = t % 2
            if t >= 1:
                out_descs[t - 1].wait()
            if t + 1 < n_steps:
                in_descs[t + 1] = pltpu.async_copy(
                    x_hbm.at[x_slice(t + 1)], xbufs[(t + 1) % 2],
                    in_sems[(t + 1) % 2])
            in_descs[t].wait()

            j = t % steps_per_b
            xb = xbufs[p]

            @plsc.parallel_loop(0, RX, 1)
            def _(r):
                @plsc.parallel_loop(0, vecs, 1, unroll=8)
                def _(c):
                    xb[r, pl.ds(c * 16, 16)] += w_buf[j * RX + r,
                                                      pl.ds(c * 16, 16)]

            out_descs[t] = pltpu.async_copy(xb, out_hbm.at[x_slice(t)],
                                            out_sems[p])
        out_descs[n_steps - 1].wait()

    return k


def kernel(x, wpe):
    B, S, D = x.shape
    return _make_sc(B, S, D)(x, wpe)


# speedup vs baseline: 3.9028x; 1.0717x over previous
import functools
import jax
import jax.numpy as jnp
from jax import lax
from jax.experimental import pallas as pl
from jax.experimental.pallas import tpu as pltpu, tpu_sc as plsc

NW = 32
RX = 32


def _make_sc(B, S, D):
    s_per_w = S // NW
    n_chunks = s_per_w // RX
    n_steps = n_chunks * B
    vecs = D // 16
    mesh = plsc.VectorSubcoreMesh(core_axis_name="c", subcore_axis_name="s")

    @functools.partial(
        pl.kernel,
        mesh=mesh,
        out_type=jax.ShapeDtypeStruct((B, S, D), jnp.float32),
        scratch_types=[
            pltpu.VMEM((RX, D), jnp.float32),
            pltpu.VMEM((RX, D), jnp.float32),
            pltpu.VMEM((RX, D), jnp.float32),
            pltpu.VMEM((RX, D), jnp.float32),
            pltpu.SemaphoreType.DMA,
            pltpu.SemaphoreType.DMA,
            pltpu.SemaphoreType.DMA,
            pltpu.SemaphoreType.DMA,
            pltpu.SemaphoreType.DMA,
            pltpu.SemaphoreType.DMA,
        ],
    )
    def k(x_hbm, wpe_hbm, out_hbm, wb0, wb1, xb0, xb1,
          sw0, sw1, si0, si1, so0, so1):
        wid = lax.axis_index("s") * 2 + lax.axis_index("c")
        s0 = wid * s_per_w
        wbufs = (wb0, wb1)
        xbufs = (xb0, xb1)
        w_sems = (sw0, sw1)
        in_sems = (si0, si1)
        out_sems = (so0, so1)

        def x_slice(t):
            ci, b = divmod(t, B)
            return (b, pl.ds(s0 + ci * RX, RX))

        w_descs = [None] * n_chunks
        in_descs = [None] * n_steps
        out_descs = [None] * n_steps

        w_descs[0] = pltpu.async_copy(wpe_hbm.at[pl.ds(s0, RX)], wbufs[0],
                                      w_sems[0])
        in_descs[0] = pltpu.async_copy(x_hbm.at[x_slice(0)], xbufs[0],
                                       in_sems[0])
        for t in range(n_steps):
            ci, b = divmod(t, B)
            p = t % 2
            if t >= 1:
                out_descs[t - 1].wait()
            if t + 1 < n_steps:
                in_descs[t + 1] = pltpu.async_copy(
                    x_hbm.at[x_slice(t + 1)], xbufs[(t + 1) % 2],
                    in_sems[(t + 1) % 2])
            if b == 0:
                if ci + 1 < n_chunks:
                    w_descs[ci + 1] = pltpu.async_copy(
                        wpe_hbm.at[pl.ds(s0 + (ci + 1) * RX, RX)],
                        wbufs[(ci + 1) % 2], w_sems[(ci + 1) % 2])
                w_descs[ci].wait()
            in_descs[t].wait()

            xb = xbufs[p]
            wb = wbufs[ci % 2]

            @plsc.parallel_loop(0, RX, 1)
            def _(r):
                @plsc.parallel_loop(0, vecs, 1, unroll=8)
                def _(c):
                    xb[r, pl.ds(c * 16, 16)] += wb[r, pl.ds(c * 16, 16)]

            out_descs[t] = pltpu.async_copy(xb, out_hbm.at[x_slice(t)],
                                            out_sems[p])
        out_descs[n_steps - 1].wait()

    return k


def kernel(x, wpe):
    B, S, D = x.shape
    return _make_sc(B, S, D)(x, wpe)
